# baseline (speedup 1.0000x reference)
import functools

import jax
import jax.numpy as jnp
from jax import lax
from jax.experimental import pallas as pl
from jax.experimental.pallas import tpu as pltpu
from jax.experimental.pallas import tpu_sc as plsc

NC = 2
NS = 16
LANES = 16
NW = NC * NS

IDX_CHUNK = 128


def _gmf_body(users_ref, items_ref, utab_ref, itab_ref, wb_ref, out_ref,
              idx_u, idx_i, rows_u, rows_i, wb_v, out_v, sem):
    bpw = rows_u.shape[0]
    n_chunks = bpw // IDX_CHUNK
    f = rows_u.shape[1]
    half = f // 2

    wid = lax.axis_index("s") * NC + lax.axis_index("c")
    base = wid * bpw

    pltpu.sync_copy(users_ref.at[wid], idx_u)
    pltpu.sync_copy(items_ref.at[wid], idx_i)
    pltpu.sync_copy(wb_ref, wb_v)

    copies = []
    for k in range(n_chunks):
        dst = rows_u.at[pl.ds(k * IDX_CHUNK, IDX_CHUNK)]
        copies.append(pltpu.async_copy(utab_ref.at[idx_u.at[k]], dst, sem))
    for k in range(n_chunks):
        dst = rows_i.at[pl.ds(k * IDX_CHUNK, IDX_CHUNK)]
        copies.append(pltpu.async_copy(itab_ref.at[idx_i.at[k]], dst, sem))
    for c in copies:
        c.wait()

    w0 = wb_v[pl.ds(0, LANES)]
    w1 = wb_v[pl.ds(half, LANES)]
    bias = wb_v[pl.ds(f, LANES)]

    iota16 = lax.iota(jnp.int32, LANES)

    def group(g, _):
        r0 = g * LANES
        z = bias
        for j in range(LANES):
            r = r0 + j
            u0 = rows_u[r, pl.ds(0, LANES)]
            u1 = rows_u[r, pl.ds(half, LANES)]
            v0 = rows_i[r, pl.ds(0, LANES)]
            v1 = rows_i[r, pl.ds(half, LANES)]
            s = jnp.sum(u0 * v0 * w0 + u1 * v1 * w1)
            z = jnp.where(iota16 == j, z + s, z)
        out_v[pl.ds(r0, LANES)] = 1.0 / (1.0 + jnp.exp(-z))
        return _

    lax.fori_loop(0, bpw // LANES, group, None)

    pltpu.sync_copy(out_v, out_ref.at[pl.ds(base, bpw)])


def kernel(users, items, user_table, item_table, pred_w, pred_b):
    b = users.shape[0]
    f = user_table.shape[1]
    assert b % (NW * IDX_CHUNK) == 0 and f == 2 * LANES
    bpw = b // NW

    users_r = users.astype(jnp.int32).reshape(NW, bpw // IDX_CHUNK, IDX_CHUNK)
    items_r = items.astype(jnp.int32).reshape(NW, bpw // IDX_CHUNK, IDX_CHUNK)
    wb = jnp.concatenate(
        [pred_w.reshape(-1), jnp.broadcast_to(pred_b.reshape(-1)[:1], (LANES,))]
    ).astype(jnp.float32)

    mesh = plsc.VectorSubcoreMesh(core_axis_name="c", subcore_axis_name="s")
    run = functools.partial(
        pl.kernel,
        out_type=jax.ShapeDtypeStruct((b,), jnp.float32),
        mesh=mesh,
        compiler_params=pltpu.CompilerParams(
            needs_layout_passes=False, use_tc_tiling_on_sc=False
        ),
        scratch_types=[
            pltpu.VMEM((bpw // IDX_CHUNK, IDX_CHUNK), jnp.int32),
            pltpu.VMEM((bpw // IDX_CHUNK, IDX_CHUNK), jnp.int32),
            pltpu.VMEM((bpw, f), jnp.float32),
            pltpu.VMEM((bpw, f), jnp.float32),
            pltpu.VMEM((f + LANES,), jnp.float32),
            pltpu.VMEM((bpw,), jnp.float32),
            pltpu.SemaphoreType.DMA,
        ],
    )(_gmf_body)
    return run(users_r, items_r, user_table, item_table, wb)
